# Initial kernel scaffold; baseline (speedup 1.0000x reference)
import jax
import jax.numpy as jnp
from jax.experimental import pallas as pl


def kernel(x, Wq, Wk_c, Wv_c, Wk, Wv, Wo, norm1_w, norm2_w, shared_in, shared_out, w1_shared, w2_expert, group_gate, expert_gate, group_bias, expert_bias):
    raise NotImplementedError("write your pallas kernel here")


# speedup vs baseline: 1.2815x; 1.2815x over previous
import functools
import math

import jax
import jax.numpy as jnp
import numpy as np
from jax.experimental import pallas as pl
from jax.experimental.pallas import tpu as pltpu

B, T, D = 1, 2048, 1024
H, DH, DC = 16, 64, 256
HID, NG, EPG, NE, TOPK = 2048, 2, 4, 8, 2
BT = 256
NBT = T // BT

_f32 = jnp.float32


def _dotT(a, b):
    return jax.lax.dot_general(a, b, (((1,), (1,)), ((), ())),
                               preferred_element_type=_f32)


def _rotary_tables():
    inv_freq = 1.0 / (10000.0 ** (np.arange(0, DH, 2, dtype=np.float64) / DH))
    pos = np.arange(T, dtype=np.float64)
    ang = np.einsum('i,j->ij', pos, inv_freq)
    cos = np.cos(ang).astype(np.float32)
    sin = np.sin(ang).astype(np.float32)
    return jnp.asarray(cos), jnp.asarray(sin)


def _head_perm():
    p = np.concatenate([np.arange(0, DH, 2), np.arange(1, DH, 2)])
    full = np.concatenate([h * DH + p for h in range(H)])
    return jnp.asarray(full, dtype=jnp.int32)


def _proj_body(x_ref, n1_ref, wq_ref, wkc_ref, wvc_ref, wk_ref, wv_ref,
               cos_ref, sin_ref, q_ref, k_ref, v_ref):
    x = x_ref[...]
    var = jnp.mean(x * x, axis=-1, keepdims=True)
    h = x / jnp.sqrt(var + 1e-6) * n1_ref[...]
    q = _dotT(h, wq_ref[...])
    kc = _dotT(h, wkc_ref[...])
    vc = _dotT(h, wvc_ref[...])
    k = _dotT(kc, wk_ref[...])
    v = _dotT(vc, wv_ref[...])
    cos = cos_ref[...]
    sin = sin_ref[...]

    for hh in range(H):
        for z, ref in ((q, q_ref), (k, k_ref)):
            x1 = z[:, hh * DH:hh * DH + DH // 2]
            x2 = z[:, hh * DH + DH // 2:(hh + 1) * DH]
            ref[hh] = jnp.concatenate(
                [x1 * cos - x2 * sin, x1 * sin + x2 * cos], axis=1)
        v_ref[hh] = v[:, hh * DH:(hh + 1) * DH]


def _proj(x2d, n1, wq_p, wkc, wvc, wk_p, wv, cos, sin):
    grid = (NBT,)
    return pl.pallas_call(
        _proj_body,
        grid=grid,
        in_specs=[
            pl.BlockSpec((BT, D), lambda i: (i, 0)),
            pl.BlockSpec((1, D), lambda i: (0, 0)),
            pl.BlockSpec((D, D), lambda i: (0, 0)),
            pl.BlockSpec((DC, D), lambda i: (0, 0)),
            pl.BlockSpec((DC, D), lambda i: (0, 0)),
            pl.BlockSpec((D, DC), lambda i: (0, 0)),
            pl.BlockSpec((D, DC), lambda i: (0, 0)),
            pl.BlockSpec((BT, DH // 2), lambda i: (i, 0)),
            pl.BlockSpec((BT, DH // 2), lambda i: (i, 0)),
        ],
        out_specs=[
            pl.BlockSpec((H, BT, DH), lambda i: (0, i, 0)),
            pl.BlockSpec((H, BT, DH), lambda i: (0, i, 0)),
            pl.BlockSpec((H, BT, DH), lambda i: (0, i, 0)),
        ],
        out_shape=[jax.ShapeDtypeStruct((H, T, DH), _f32)] * 3,
    )(x2d, n1, wq_p, wkc, wvc, wk_p, wv, cos, sin)


def _attn_body(q_ref, k_ref, v_ref, o_ref):
    q = q_ref[0]
    s = jax.lax.dot_general(q, k_ref[0], (((1,), (1,)), ((), ())),
                            preferred_element_type=_f32)
    s = s * (1.0 / math.sqrt(DH))
    m = jnp.max(s, axis=-1, keepdims=True)
    p = jnp.exp(s - m)
    l = jnp.sum(p, axis=-1, keepdims=True)
    o = jax.lax.dot_general(p, v_ref[0], (((1,), (0,)), ((), ())),
                            preferred_element_type=_f32)
    o_ref[0] = o / l


def _attn(q, k, v):
    grid = (H, NBT)
    return pl.pallas_call(
        _attn_body,
        grid=grid,
        in_specs=[
            pl.BlockSpec((1, BT, DH), lambda h, i: (h, i, 0)),
            pl.BlockSpec((1, T, DH), lambda h, i: (h, 0, 0)),
            pl.BlockSpec((1, T, DH), lambda h, i: (h, 0, 0)),
        ],
        out_specs=pl.BlockSpec((1, BT, DH), lambda h, i: (h, i, 0)),
        out_shape=jax.ShapeDtypeStruct((H, T, DH), _f32),
    )(q, k, v)


def _post_body(o_ref, x_ref, wo_ref, n2_ref, xa_ref, h2_ref):
    o2 = jnp.concatenate([o_ref[hh] for hh in range(H)], axis=1)
    xa = x_ref[...] + _dotT(o2, wo_ref[...])
    xa_ref[...] = xa
    var = jnp.mean(xa * xa, axis=-1, keepdims=True)
    h2_ref[...] = xa / jnp.sqrt(var + 1e-6) * n2_ref[...]


def _post(o, x2d, wo, n2):
    return pl.pallas_call(
        _post_body,
        grid=(NBT,),
        in_specs=[
            pl.BlockSpec((H, BT, DH), lambda i: (0, i, 0)),
            pl.BlockSpec((BT, D), lambda i: (i, 0)),
            pl.BlockSpec((D, D), lambda i: (0, 0)),
            pl.BlockSpec((1, D), lambda i: (0, 0)),
        ],
        out_specs=[
            pl.BlockSpec((BT, D), lambda i: (i, 0)),
            pl.BlockSpec((BT, D), lambda i: (i, 0)),
        ],
        out_shape=[jax.ShapeDtypeStruct((T, D), _f32)] * 2,
    )(o, x2d, wo, n2)


def _gates_body(h2_ref, gg_ref, eg_ref, gb_ref, eb_ref, wgt_ref):
    h2 = h2_ref[...]
    glog = _dotT(h2, gg_ref[...]) + gb_ref[...]
    gm = jnp.max(glog, axis=-1, keepdims=True)
    ge = jnp.exp(glog - gm)
    gprobs = ge / jnp.sum(ge, axis=-1, keepdims=True)
    g_idx = (gprobs[:, 1:2] > gprobs[:, 0:1]).astype(jnp.int32)
    g_prob = jnp.max(gprobs, axis=-1, keepdims=True)

    elog = _dotT(h2, eg_ref[...]) + eb_ref[...]
    idx8 = jax.lax.broadcasted_iota(jnp.int32, (BT, NE), 1)
    allowed = (idx8 // EPG) == g_idx
    masked = jnp.where(allowed, elog, -jnp.inf)
    m = jnp.max(masked, axis=-1, keepdims=True)
    ex = jnp.exp(masked - m)
    eprobs = ex / jnp.sum(ex, axis=-1, keepdims=True)
    p = eprobs * g_prob

    m1 = jnp.max(p, axis=-1, keepdims=True)
    i1 = jnp.min(jnp.where(p == m1, idx8, NE), axis=-1, keepdims=True)
    p2 = jnp.where(idx8 == i1, -1.0, p)
    m2 = jnp.max(p2, axis=-1, keepdims=True)
    i2 = jnp.min(jnp.where(p2 == m2, idx8, NE), axis=-1, keepdims=True)
    wgt = jnp.where(idx8 == i1, m1, 0.0) + jnp.where(idx8 == i2, m2, 0.0)
    wgt_ref[...] = wgt


def _gates(h2, gg, eg, gb, eb):
    return pl.pallas_call(
        _gates_body,
        grid=(NBT,),
        in_specs=[
            pl.BlockSpec((BT, D), lambda i: (i, 0)),
            pl.BlockSpec((NG, D), lambda i: (0, 0)),
            pl.BlockSpec((NE, D), lambda i: (0, 0)),
            pl.BlockSpec((1, NG), lambda i: (0, 0)),
            pl.BlockSpec((1, NE), lambda i: (0, 0)),
        ],
        out_specs=pl.BlockSpec((BT, NE), lambda i: (i, 0)),
        out_shape=jax.ShapeDtypeStruct((T, NE), _f32),
    )(h2, gg, eg, gb, eb)


def _swiglu(z):
    a = z[:, :HID]
    b = z[:, HID:]
    return a * jax.lax.logistic(a) * b


def _ffn_up_body(h2_ref, w_ref, out_ref):
    out_ref[...] = _swiglu(_dotT(h2_ref[...], w_ref[...]))


def _ffn_up(h2, w):
    return pl.pallas_call(
        _ffn_up_body,
        grid=(NBT,),
        in_specs=[
            pl.BlockSpec((BT, D), lambda i: (i, 0)),
            pl.BlockSpec((2 * HID, D), lambda i: (0, 0)),
        ],
        out_specs=pl.BlockSpec((BT, HID), lambda i: (i, 0)),
        out_shape=jax.ShapeDtypeStruct((T, HID), _f32),
    )(h2, w)


def _routed_body(hm_ref, w2_ref, wgt_ref, out_ref, acc_ref):
    e = pl.program_id(0)
    i = pl.program_id(1)
    idx8 = jax.lax.broadcasted_iota(jnp.int32, (BT, NE), 1)
    wcol = jnp.sum(jnp.where(idx8 == e, wgt_ref[...], 0.0), axis=1,
                   keepdims=True)
    y = jax.lax.dot_general(hm_ref[...], w2_ref[0], (((1,), (1,)), ((), ())),
                            preferred_element_type=_f32) * wcol
    sl = pl.ds(i * BT, BT)

    @pl.when(e == 0)
    def _():
        acc_ref[sl, :] = y

    @pl.when(e > 0)
    def _():
        acc_ref[sl, :] += y

    out_ref[...] = acc_ref[sl, :]


def _routed(h_mid, w2, wgt):
    return pl.pallas_call(
        _routed_body,
        grid=(NE, NBT),
        in_specs=[
            pl.BlockSpec((BT, HID), lambda e, i: (i, 0)),
            pl.BlockSpec((1, D, HID), lambda e, i: (e, 0, 0)),
            pl.BlockSpec((BT, NE), lambda e, i: (i, 0)),
        ],
        out_specs=pl.BlockSpec((BT, D), lambda e, i: (i, 0)),
        out_shape=jax.ShapeDtypeStruct((T, D), _f32),
        scratch_shapes=[pltpu.VMEM((T, D), _f32)],
    )(h_mid, w2, wgt)


def _final_body(xa_ref, hid1_ref, so_ref, routed_ref, out_ref):
    sh = _dotT(hid1_ref[...], so_ref[...])
    out_ref[...] = xa_ref[...] + sh + routed_ref[...]


def _final(xa, hid1, so, routed):
    return pl.pallas_call(
        _final_body,
        grid=(NBT,),
        in_specs=[
            pl.BlockSpec((BT, D), lambda i: (i, 0)),
            pl.BlockSpec((BT, HID), lambda i: (i, 0)),
            pl.BlockSpec((D, HID), lambda i: (0, 0)),
            pl.BlockSpec((BT, D), lambda i: (i, 0)),
        ],
        out_specs=pl.BlockSpec((BT, D), lambda i: (i, 0)),
        out_shape=jax.ShapeDtypeStruct((T, D), _f32),
    )(xa, hid1, so, routed)


def kernel(x, Wq, Wk_c, Wv_c, Wk, Wv, Wo, norm1_w, norm2_w, shared_in,
           shared_out, w1_shared, w2_expert, group_gate, expert_gate,
           group_bias, expert_bias):
    x2d = x.reshape(T, D)
    perm = _head_perm()
    wq_p = Wq[perm, :]
    wk_p = Wk[perm, :]
    cos, sin = _rotary_tables()

    q, k, v = _proj(x2d, norm1_w.reshape(1, D), wq_p, Wk_c, Wv_c, wk_p, Wv,
                    cos, sin)
    o = _attn(q, k, v)
    xa, h2 = _post(o, x2d, Wo, norm2_w.reshape(1, D))

    wgt = _gates(h2, group_gate, expert_gate, group_bias.reshape(1, NG),
                 expert_bias.reshape(1, NE))
    hid1 = _ffn_up(h2, shared_in)
    h_mid = _ffn_up(h2, w1_shared)
    routed = _routed(h_mid, w2_expert, wgt)
    out = _final(xa, hid1, shared_out, routed)
    return out.reshape(B, T, D)


# speedup vs baseline: 1.4184x; 1.1068x over previous
import functools
import math

import jax
import jax.numpy as jnp
import numpy as np
from jax.experimental import pallas as pl
from jax.experimental.pallas import tpu as pltpu

B, T, D = 1, 2048, 1024
H, DH, DC = 16, 64, 256
HID, NG, EPG, NE, TOPK = 2048, 2, 4, 8, 2
BT = 256
NBT = T // BT

_f32 = jnp.float32


def _dotT(a, b):
    return jax.lax.dot_general(a, b, (((1,), (1,)), ((), ())),
                               preferred_element_type=_f32)


def _rotary_tables():
    inv_freq = 1.0 / (10000.0 ** (np.arange(0, DH, 2, dtype=np.float64) / DH))
    pos = np.arange(T, dtype=np.float64)
    ang = np.einsum('i,j->ij', pos, inv_freq)
    cos = np.cos(ang).astype(np.float32)
    sin = np.sin(ang).astype(np.float32)
    return jnp.asarray(cos), jnp.asarray(sin)


def _head_perm():
    p = np.concatenate([np.arange(0, DH, 2), np.arange(1, DH, 2)])
    full = np.concatenate([h * DH + p for h in range(H)])
    return jnp.asarray(full, dtype=jnp.int32)


def _proj_body(x_ref, n1_ref, wq_ref, wkc_ref, wvc_ref, wk_ref, wv_ref,
               cos_ref, sin_ref, q_ref, k_ref, v_ref):
    x = x_ref[...]
    var = jnp.mean(x * x, axis=-1, keepdims=True)
    h = x / jnp.sqrt(var + 1e-6) * n1_ref[...]
    q = _dotT(h, wq_ref[...])
    kc = _dotT(h, wkc_ref[...])
    vc = _dotT(h, wvc_ref[...])
    k = _dotT(kc, wk_ref[...])
    v = _dotT(vc, wv_ref[...])
    cos = cos_ref[...]
    sin = sin_ref[...]

    for hh in range(H):
        for z, ref in ((q, q_ref), (k, k_ref)):
            x1 = z[:, hh * DH:hh * DH + DH // 2]
            x2 = z[:, hh * DH + DH // 2:(hh + 1) * DH]
            ref[hh] = jnp.concatenate(
                [x1 * cos - x2 * sin, x1 * sin + x2 * cos], axis=1)
        v_ref[hh] = v[:, hh * DH:(hh + 1) * DH]


def _proj(x2d, n1, wq_p, wkc, wvc, wk_p, wv, cos, sin):
    grid = (NBT,)
    return pl.pallas_call(
        _proj_body,
        grid=grid,
        in_specs=[
            pl.BlockSpec((BT, D), lambda i: (i, 0)),
            pl.BlockSpec((1, D), lambda i: (0, 0)),
            pl.BlockSpec((D, D), lambda i: (0, 0)),
            pl.BlockSpec((DC, D), lambda i: (0, 0)),
            pl.BlockSpec((DC, D), lambda i: (0, 0)),
            pl.BlockSpec((D, DC), lambda i: (0, 0)),
            pl.BlockSpec((D, DC), lambda i: (0, 0)),
            pl.BlockSpec((BT, DH // 2), lambda i: (i, 0)),
            pl.BlockSpec((BT, DH // 2), lambda i: (i, 0)),
        ],
        out_specs=[
            pl.BlockSpec((H, BT, DH), lambda i: (0, i, 0)),
            pl.BlockSpec((H, BT, DH), lambda i: (0, i, 0)),
            pl.BlockSpec((H, BT, DH), lambda i: (0, i, 0)),
        ],
        out_shape=[jax.ShapeDtypeStruct((H, T, DH), _f32)] * 3,
    )(x2d, n1, wq_p, wkc, wvc, wk_p, wv, cos, sin)


def _attn_body(q_ref, k_ref, v_ref, o_ref):
    q = q_ref[0]
    s = jax.lax.dot_general(q, k_ref[0], (((1,), (1,)), ((), ())),
                            preferred_element_type=_f32)
    s = s * (1.0 / math.sqrt(DH))
    m = jnp.max(s, axis=-1, keepdims=True)
    p = jnp.exp(s - m)
    l = jnp.sum(p, axis=-1, keepdims=True)
    o = jax.lax.dot_general(p, v_ref[0], (((1,), (0,)), ((), ())),
                            preferred_element_type=_f32)
    o_ref[0] = o / l


def _attn(q, k, v):
    grid = (H, NBT)
    return pl.pallas_call(
        _attn_body,
        grid=grid,
        in_specs=[
            pl.BlockSpec((1, BT, DH), lambda h, i: (h, i, 0)),
            pl.BlockSpec((1, T, DH), lambda h, i: (h, 0, 0)),
            pl.BlockSpec((1, T, DH), lambda h, i: (h, 0, 0)),
        ],
        out_specs=pl.BlockSpec((1, BT, DH), lambda h, i: (h, i, 0)),
        out_shape=jax.ShapeDtypeStruct((H, T, DH), _f32),
    )(q, k, v)


def _post_body(o_ref, x_ref, wo_ref, n2_ref, xa_ref, h2_ref):
    o2 = jnp.concatenate([o_ref[hh] for hh in range(H)], axis=1)
    xa = x_ref[...] + _dotT(o2, wo_ref[...])
    xa_ref[...] = xa
    var = jnp.mean(xa * xa, axis=-1, keepdims=True)
    h2_ref[...] = xa / jnp.sqrt(var + 1e-6) * n2_ref[...]


def _post(o, x2d, wo, n2):
    return pl.pallas_call(
        _post_body,
        grid=(NBT,),
        in_specs=[
            pl.BlockSpec((H, BT, DH), lambda i: (0, i, 0)),
            pl.BlockSpec((BT, D), lambda i: (i, 0)),
            pl.BlockSpec((D, D), lambda i: (0, 0)),
            pl.BlockSpec((1, D), lambda i: (0, 0)),
        ],
        out_specs=[
            pl.BlockSpec((BT, D), lambda i: (i, 0)),
            pl.BlockSpec((BT, D), lambda i: (i, 0)),
        ],
        out_shape=[jax.ShapeDtypeStruct((T, D), _f32)] * 2,
    )(o, x2d, wo, n2)


def _gates_body(h2_ref, gg_ref, eg_ref, gb_ref, eb_ref, wgt_ref):
    h2 = h2_ref[...]
    glog = _dotT(h2, gg_ref[...]) + gb_ref[...]
    gm = jnp.max(glog, axis=-1, keepdims=True)
    ge = jnp.exp(glog - gm)
    gprobs = ge / jnp.sum(ge, axis=-1, keepdims=True)
    g_idx = (gprobs[:, 1:2] > gprobs[:, 0:1]).astype(jnp.int32)
    g_prob = jnp.max(gprobs, axis=-1, keepdims=True)

    elog = _dotT(h2, eg_ref[...]) + eb_ref[...]
    idx8 = jax.lax.broadcasted_iota(jnp.int32, (BT, NE), 1)
    allowed = (idx8 // EPG) == g_idx
    masked = jnp.where(allowed, elog, -jnp.inf)
    m = jnp.max(masked, axis=-1, keepdims=True)
    ex = jnp.exp(masked - m)
    eprobs = ex / jnp.sum(ex, axis=-1, keepdims=True)
    p = eprobs * g_prob

    m1 = jnp.max(p, axis=-1, keepdims=True)
    i1 = jnp.min(jnp.where(p == m1, idx8, NE), axis=-1, keepdims=True)
    p2 = jnp.where(idx8 == i1, -1.0, p)
    m2 = jnp.max(p2, axis=-1, keepdims=True)
    i2 = jnp.min(jnp.where(p2 == m2, idx8, NE), axis=-1, keepdims=True)
    wgt = jnp.where(idx8 == i1, m1, 0.0) + jnp.where(idx8 == i2, m2, 0.0)
    wgt_ref[...] = wgt


def _gates(h2, gg, eg, gb, eb):
    return pl.pallas_call(
        _gates_body,
        grid=(NBT,),
        in_specs=[
            pl.BlockSpec((BT, D), lambda i: (i, 0)),
            pl.BlockSpec((NG, D), lambda i: (0, 0)),
            pl.BlockSpec((NE, D), lambda i: (0, 0)),
            pl.BlockSpec((1, NG), lambda i: (0, 0)),
            pl.BlockSpec((1, NE), lambda i: (0, 0)),
        ],
        out_specs=pl.BlockSpec((BT, NE), lambda i: (i, 0)),
        out_shape=jax.ShapeDtypeStruct((T, NE), _f32),
    )(h2, gg, eg, gb, eb)


def _swiglu(z):
    a = z[:, :HID]
    b = z[:, HID:]
    return a * jax.lax.logistic(a) * b


def _ffn_up_body(h2_ref, w_ref, out_ref):
    out_ref[...] = _swiglu(_dotT(h2_ref[...], w_ref[...]))


def _ffn_up(h2, w):
    return pl.pallas_call(
        _ffn_up_body,
        grid=(NBT,),
        in_specs=[
            pl.BlockSpec((BT, D), lambda i: (i, 0)),
            pl.BlockSpec((2 * HID, D), lambda i: (0, 0)),
        ],
        out_specs=pl.BlockSpec((BT, HID), lambda i: (i, 0)),
        out_shape=jax.ShapeDtypeStruct((T, HID), _f32),
    )(h2, w)


def _routed_body(hm_ref, w2_ref, wgt_ref, out_ref, acc_ref):
    e = pl.program_id(0)
    idx8 = jax.lax.broadcasted_iota(jnp.int32, (T, NE), 1)
    wcol = jnp.sum(jnp.where(idx8 == e, wgt_ref[...], 0.0), axis=1,
                   keepdims=True)
    y = jax.lax.dot_general(hm_ref[...], w2_ref[0], (((1,), (1,)), ((), ())),
                            preferred_element_type=_f32) * wcol

    @pl.when(e == 0)
    def _():
        acc_ref[...] = y

    @pl.when(e > 0)
    def _():
        acc_ref[...] += y

    @pl.when(e == NE - 1)
    def _():
        out_ref[...] = acc_ref[...]


def _routed(h_mid, w2, wgt):
    return pl.pallas_call(
        _routed_body,
        grid=(NE,),
        in_specs=[
            pl.BlockSpec((T, HID), lambda e: (0, 0)),
            pl.BlockSpec((1, D, HID), lambda e: (e, 0, 0)),
            pl.BlockSpec((T, NE), lambda e: (0, 0)),
        ],
        out_specs=pl.BlockSpec((T, D), lambda e: (0, 0)),
        out_shape=jax.ShapeDtypeStruct((T, D), _f32),
        scratch_shapes=[pltpu.VMEM((T, D), _f32)],
    )(h_mid, w2, wgt)


def _final_body(xa_ref, hid1_ref, so_ref, routed_ref, out_ref):
    sh = _dotT(hid1_ref[...], so_ref[...])
    out_ref[...] = xa_ref[...] + sh + routed_ref[...]


def _final(xa, hid1, so, routed):
    return pl.pallas_call(
        _final_body,
        grid=(NBT,),
        in_specs=[
            pl.BlockSpec((BT, D), lambda i: (i, 0)),
            pl.BlockSpec((BT, HID), lambda i: (i, 0)),
            pl.BlockSpec((D, HID), lambda i: (0, 0)),
            pl.BlockSpec((BT, D), lambda i: (i, 0)),
        ],
        out_specs=pl.BlockSpec((BT, D), lambda i: (i, 0)),
        out_shape=jax.ShapeDtypeStruct((T, D), _f32),
    )(xa, hid1, so, routed)


def kernel(x, Wq, Wk_c, Wv_c, Wk, Wv, Wo, norm1_w, norm2_w, shared_in,
           shared_out, w1_shared, w2_expert, group_gate, expert_gate,
           group_bias, expert_bias):
    x2d = x.reshape(T, D)
    perm = _head_perm()
    wq_p = Wq[perm, :]
    wk_p = Wk[perm, :]
    cos, sin = _rotary_tables()

    q, k, v = _proj(x2d, norm1_w.reshape(1, D), wq_p, Wk_c, Wv_c, wk_p, Wv,
                    cos, sin)
    o = _attn(q, k, v)
    xa, h2 = _post(o, x2d, Wo, norm2_w.reshape(1, D))

    wgt = _gates(h2, group_gate, expert_gate, group_bias.reshape(1, NG),
                 expert_bias.reshape(1, NE))
    hid1 = _ffn_up(h2, shared_in)
    h_mid = _ffn_up(h2, w1_shared)
    routed = _routed(h_mid, w2_expert, wgt)
    out = _final(xa, hid1, shared_out, routed)
    return out.reshape(B, T, D)
